# baseline (device time: 139706 ns/iter reference)
import jax
import jax.numpy as jnp
from jax import lax
from jax.experimental import pallas as pl
from jax.experimental.pallas import tpu as pltpu

N_DEV = 4
E_LOC = 4
E_GLB = 16
T_LOC = 256
D = 1024
F = 2048


def kernel(x, router, W1, W2):
    def body(x_ref, r_ref, w1_hbm, w2_hbm, out_ref,
             xfull, rfull, wfull, w1buf, w2buf, partial, rs_send, rs_recv,
             xs_send, xs_recv, rt_send, rt_recv, wi_send, wi_recv,
             os_send, os_recv, w1sem, w2sem):
        my = lax.axis_index("i")
        right = lax.rem(my + 1, N_DEV)
        left = lax.rem(my + N_DEV - 1, N_DEV)

        cp1 = pltpu.make_async_copy(w1_hbm.at[0], w1buf, w1sem)
        cp2 = pltpu.make_async_copy(w2_hbm.at[0], w2buf, w2sem)
        cp1.start()
        cp2.start()

        xfull[pl.ds(my, 1)] = x_ref[...].astype(jnp.bfloat16)[None]
        rfull[pl.ds(my, 1)] = r_ref[...][None]

        barrier_sem = pltpu.get_barrier_semaphore()
        for nbr in (left, right):
            pl.semaphore_signal(barrier_sem, inc=1, device_id=(nbr,),
                                device_id_type=pl.DeviceIdType.MESH)
        pl.semaphore_wait(barrier_sem, 2)

        for h in range(N_DEV - 1):
            slot = lax.rem(my - h + N_DEV, N_DEV)
            rdma = pltpu.make_async_remote_copy(
                src_ref=rfull.at[slot], dst_ref=rfull.at[slot],
                send_sem=rt_send.at[h], recv_sem=rt_recv.at[h],
                device_id=(right,), device_id_type=pl.DeviceIdType.MESH)
            rdma.start()
            rdma.wait()

        rcat = jnp.concatenate([rfull[j] for j in range(N_DEV)], axis=1)
        gates = lax.dot_general(
            x_ref[...], rcat, (((1,), (0,)), ((), ())),
            precision=lax.Precision.HIGHEST,
            preferred_element_type=jnp.float32)
        iota = lax.broadcasted_iota(jnp.int32, (T_LOC, E_GLB), 1)
        v1 = jnp.max(gates, axis=1, keepdims=True)
        a1 = jnp.min(jnp.where(gates == v1, iota, E_GLB), axis=1, keepdims=True)
        masked = jnp.where(iota == a1, -jnp.inf, gates)
        v2 = jnp.max(masked, axis=1, keepdims=True)
        a2 = jnp.min(jnp.where(masked == v2, iota, E_GLB), axis=1, keepdims=True)
        e2 = jnp.exp(v2 - v1)
        w1n = 1.0 / (1.0 + e2)
        w2n = e2 / (1.0 + e2)
        winfo = jnp.concatenate(
            [a1.astype(jnp.float32), a2.astype(jnp.float32), w1n, w2n], axis=1)
        wfull[pl.ds(my, 1)] = winfo[None]

        for h in range(N_DEV - 1):
            slot = lax.rem(my - h + N_DEV, N_DEV)
            rdma_x = pltpu.make_async_remote_copy(
                src_ref=xfull.at[slot], dst_ref=xfull.at[slot],
                send_sem=xs_send.at[h], recv_sem=xs_recv.at[h],
                device_id=(right,), device_id_type=pl.DeviceIdType.MESH)
            rdma_w = pltpu.make_async_remote_copy(
                src_ref=wfull.at[slot], dst_ref=wfull.at[slot],
                send_sem=wi_send.at[h], recv_sem=wi_recv.at[h],
                device_id=(right,), device_id_type=pl.DeviceIdType.MESH)
            rdma_x.start()
            rdma_w.start()
            rdma_x.wait()
            rdma_w.wait()

        xall = xfull[...].reshape(N_DEV * T_LOC, D).astype(jnp.float32)
        wf = wfull[...].reshape(N_DEV * T_LOC, 4)
        a1c, a2c = wf[:, 0:1], wf[:, 1:2]
        w1c, w2c = wf[:, 2:3], wf[:, 3:4]

        acc = jnp.zeros((N_DEV * T_LOC, D), jnp.float32)
        for e in range(E_LOC):
            if e > 0:
                pltpu.make_async_copy(w1_hbm.at[e], w1buf, w1sem).start()
                pltpu.make_async_copy(w2_hbm.at[e], w2buf, w2sem).start()
            pltpu.make_async_copy(w1_hbm.at[e], w1buf, w1sem).wait()
            pltpu.make_async_copy(w2_hbm.at[e], w2buf, w2sem).wait()
            h_ = jnp.maximum(
                jnp.dot(xall, w1buf[...],
                        preferred_element_type=jnp.float32),
                0.0)
            p = jnp.dot(h_, w2buf[...], preferred_element_type=jnp.float32)
            g_id = (my * E_LOC + e).astype(jnp.float32)
            wcol = (jnp.where(a1c == g_id, w1c, 0.0)
                    + jnp.where(a2c == g_id, w2c, 0.0))
            acc = acc + p * wcol
        partial[...] = acc.reshape(N_DEV, T_LOC, D)

        for h in range(N_DEV - 1):
            c = lax.rem(my - 1 - h + 2 * N_DEV, N_DEV)
            chunk = partial[pl.ds(c, 1)].reshape(T_LOC, D)
            if h == 0:
                val = chunk
            else:
                val = chunk + rs_recv[h - 1].astype(jnp.float32)
            rs_send[...] = val.astype(jnp.bfloat16)
            rdma = pltpu.make_async_remote_copy(
                src_ref=rs_send, dst_ref=rs_recv.at[h],
                send_sem=os_send.at[h], recv_sem=os_recv.at[h],
                device_id=(right,), device_id_type=pl.DeviceIdType.MESH)
            rdma.start()
            rdma.wait()
        out_ref[...] = (partial[pl.ds(my, 1)].reshape(T_LOC, D)
                        + rs_recv[N_DEV - 2].astype(jnp.float32))

    return pl.pallas_call(
        body,
        out_shape=jax.ShapeDtypeStruct((T_LOC, D), jnp.float32),
        in_specs=[
            pl.BlockSpec(memory_space=pltpu.VMEM),
            pl.BlockSpec(memory_space=pltpu.VMEM),
            pl.BlockSpec(memory_space=pltpu.MemorySpace.HBM),
            pl.BlockSpec(memory_space=pltpu.MemorySpace.HBM),
        ],
        out_specs=pl.BlockSpec(memory_space=pltpu.VMEM),
        scratch_shapes=[
            pltpu.VMEM((N_DEV, T_LOC, D), jnp.bfloat16),
            pltpu.VMEM((N_DEV, D, E_LOC), jnp.float32),
            pltpu.VMEM((N_DEV, T_LOC, 4), jnp.float32),
            pltpu.VMEM((D, F), jnp.float32),
            pltpu.VMEM((F, D), jnp.float32),
            pltpu.VMEM((N_DEV, T_LOC, D), jnp.float32),
            pltpu.VMEM((T_LOC, D), jnp.bfloat16),
            pltpu.VMEM((N_DEV - 1, T_LOC, D), jnp.bfloat16),
            pltpu.SemaphoreType.DMA((N_DEV - 1,)),
            pltpu.SemaphoreType.DMA((N_DEV - 1,)),
            pltpu.SemaphoreType.DMA((N_DEV - 1,)),
            pltpu.SemaphoreType.DMA((N_DEV - 1,)),
            pltpu.SemaphoreType.DMA((N_DEV - 1,)),
            pltpu.SemaphoreType.DMA((N_DEV - 1,)),
            pltpu.SemaphoreType.DMA((N_DEV - 1,)),
            pltpu.SemaphoreType.DMA((N_DEV - 1,)),
            pltpu.SemaphoreType.DMA,
            pltpu.SemaphoreType.DMA,
        ],
        compiler_params=pltpu.CompilerParams(
            collective_id=0, vmem_limit_bytes=64 * 1024 * 1024),
    )(x, router, W1, W2)


# device time: 94244 ns/iter; 1.4824x vs baseline; 1.4824x over previous
import jax
import jax.numpy as jnp
from jax import lax
from jax.experimental import pallas as pl
from jax.experimental.pallas import tpu as pltpu

N_DEV = 4
E_LOC = 4
E_GLB = 16
T_LOC = 256
D = 1024
F = 2048
H = F // 2
N_STAGE = E_LOC * 2


def kernel(x, router, W1, W2):
    def body(x_ref, r_ref, w1_hbm, w2_hbm, out_ref,
             xfull, rfull, wfull, w1buf, w2buf, partial, rs_out, rs_in,
             xg_send, xg_recv, rt_send, rt_recv, wi_send, wi_recv,
             rs_ssem, rs_rsem, w1sem, w2sem):
        my = lax.axis_index("i")

        def wcopies(s):
            e, hf = divmod(s, 2)
            sl = s % 2
            return (
                pltpu.make_async_copy(
                    w1_hbm.at[e, :, pl.ds(hf * H, H)], w1buf.at[sl],
                    w1sem.at[sl]),
                pltpu.make_async_copy(
                    w2_hbm.at[e, pl.ds(hf * H, H), :], w2buf.at[sl],
                    w2sem.at[sl]),
            )

        for s in (0, 1):
            for cp in wcopies(s):
                cp.start()

        xfull[pl.ds(my, 1)] = x_ref[...].astype(jnp.bfloat16)[None]
        rfull[pl.ds(my, 1)] = r_ref[...][None]

        barrier_sem = pltpu.get_barrier_semaphore()
        for k in range(1, N_DEV):
            pl.semaphore_signal(barrier_sem, inc=1,
                                device_id=(lax.rem(my + k, N_DEV),),
                                device_id_type=pl.DeviceIdType.MESH)
        pl.semaphore_wait(barrier_sem, N_DEV - 1)

        for k in range(1, N_DEV):
            dst = lax.rem(my + k, N_DEV)
            pltpu.make_async_remote_copy(
                src_ref=xfull.at[my], dst_ref=xfull.at[my],
                send_sem=xg_send.at[k - 1], recv_sem=xg_recv.at[my],
                device_id=(dst,),
                device_id_type=pl.DeviceIdType.MESH).start()
            pltpu.make_async_remote_copy(
                src_ref=rfull.at[my], dst_ref=rfull.at[my],
                send_sem=rt_send.at[k - 1], recv_sem=rt_recv.at[my],
                device_id=(dst,),
                device_id_type=pl.DeviceIdType.MESH).start()

        for k in range(1, N_DEV):
            s = lax.rem(my + k, N_DEV)
            pltpu.make_async_remote_copy(
                src_ref=rfull.at[s], dst_ref=rfull.at[s],
                send_sem=rt_send.at[0], recv_sem=rt_recv.at[s],
                device_id=(my,),
                device_id_type=pl.DeviceIdType.MESH).wait_recv()

        rcat = jnp.concatenate([rfull[j] for j in range(N_DEV)], axis=1)
        gates = lax.dot_general(
            x_ref[...], rcat, (((1,), (0,)), ((), ())),
            precision=lax.Precision.HIGHEST,
            preferred_element_type=jnp.float32)
        iota = lax.broadcasted_iota(jnp.int32, (T_LOC, E_GLB), 1)
        v1 = jnp.max(gates, axis=1, keepdims=True)
        a1 = jnp.min(jnp.where(gates == v1, iota, E_GLB), axis=1, keepdims=True)
        masked = jnp.where(iota == a1, -jnp.inf, gates)
        v2 = jnp.max(masked, axis=1, keepdims=True)
        a2 = jnp.min(jnp.where(masked == v2, iota, E_GLB), axis=1, keepdims=True)
        e2 = jnp.exp(v2 - v1)
        w1n = 1.0 / (1.0 + e2)
        w2n = e2 / (1.0 + e2)
        winfo = jnp.concatenate(
            [a1.astype(jnp.float32), a2.astype(jnp.float32), w1n, w2n], axis=1)
        wfull[pl.ds(my, 1)] = winfo[None]

        for k in range(1, N_DEV):
            dst = lax.rem(my + k, N_DEV)
            pltpu.make_async_remote_copy(
                src_ref=wfull.at[my], dst_ref=wfull.at[my],
                send_sem=wi_send.at[k - 1], recv_sem=wi_recv.at[my],
                device_id=(dst,),
                device_id_type=pl.DeviceIdType.MESH).start()

        for k in range(1, N_DEV):
            s = lax.rem(my + k, N_DEV)
            pltpu.make_async_remote_copy(
                src_ref=xfull.at[s], dst_ref=xfull.at[s],
                send_sem=xg_send.at[0], recv_sem=xg_recv.at[s],
                device_id=(my,),
                device_id_type=pl.DeviceIdType.MESH).wait_recv()
            pltpu.make_async_remote_copy(
                src_ref=wfull.at[s], dst_ref=wfull.at[s],
                send_sem=wi_send.at[0], recv_sem=wi_recv.at[s],
                device_id=(my,),
                device_id_type=pl.DeviceIdType.MESH).wait_recv()

        xall = xfull[...].reshape(N_DEV * T_LOC, D).astype(jnp.float32)
        wf = wfull[...].reshape(N_DEV * T_LOC, 4)
        a1c, a2c = wf[:, 0:1], wf[:, 1:2]
        w1c, w2c = wf[:, 2:3], wf[:, 3:4]

        acc = jnp.zeros((N_DEV * T_LOC, D), jnp.float32)
        for s in range(N_STAGE):
            e = s // 2
            for cp in wcopies(s):
                cp.wait()
            hh = jnp.maximum(
                jnp.dot(xall, w1buf[s % 2],
                        preferred_element_type=jnp.float32), 0.0)
            ph = jnp.dot(hh, w2buf[s % 2],
                         preferred_element_type=jnp.float32)
            if s + 2 < N_STAGE:
                for cp in wcopies(s + 2):
                    cp.start()
            g_id = (my * E_LOC + e).astype(jnp.float32)
            wcol = (jnp.where(a1c == g_id, w1c, 0.0)
                    + jnp.where(a2c == g_id, w2c, 0.0))
            acc = acc + ph * wcol
        partial[...] = acc.reshape(N_DEV, T_LOC, D)

        for k in range(1, N_DEV):
            dst = lax.rem(my + k, N_DEV)
            rs_out[pl.ds(k - 1, 1)] = partial[pl.ds(dst, 1)].astype(jnp.bfloat16)
            pltpu.make_async_remote_copy(
                src_ref=rs_out.at[k - 1], dst_ref=rs_in.at[my],
                send_sem=rs_ssem.at[k - 1], recv_sem=rs_rsem.at[my],
                device_id=(dst,),
                device_id_type=pl.DeviceIdType.MESH).start()
        res = partial[pl.ds(my, 1)].reshape(T_LOC, D)
        for k in range(1, N_DEV):
            s = lax.rem(my + k, N_DEV)
            pltpu.make_async_remote_copy(
                src_ref=rs_in.at[s], dst_ref=rs_in.at[s],
                send_sem=rs_ssem.at[0], recv_sem=rs_rsem.at[s],
                device_id=(my,),
                device_id_type=pl.DeviceIdType.MESH).wait_recv()
            res = res + rs_in[pl.ds(s, 1)].reshape(T_LOC, D).astype(jnp.float32)
        out_ref[...] = res

        for k in range(1, N_DEV):
            dst = lax.rem(my + k, N_DEV)
            for src, ssem in ((xfull.at[my], xg_send.at[k - 1]),
                              (rfull.at[my], rt_send.at[k - 1]),
                              (wfull.at[my], wi_send.at[k - 1]),
                              (rs_out.at[k - 1], rs_ssem.at[k - 1])):
                pltpu.make_async_remote_copy(
                    src_ref=src, dst_ref=src,
                    send_sem=ssem, recv_sem=xg_recv.at[my],
                    device_id=(dst,),
                    device_id_type=pl.DeviceIdType.MESH).wait_send()

    return pl.pallas_call(
        body,
        out_shape=jax.ShapeDtypeStruct((T_LOC, D), jnp.float32),
        in_specs=[
            pl.BlockSpec(memory_space=pltpu.VMEM),
            pl.BlockSpec(memory_space=pltpu.VMEM),
            pl.BlockSpec(memory_space=pltpu.MemorySpace.HBM),
            pl.BlockSpec(memory_space=pltpu.MemorySpace.HBM),
        ],
        out_specs=pl.BlockSpec(memory_space=pltpu.VMEM),
        scratch_shapes=[
            pltpu.VMEM((N_DEV, T_LOC, D), jnp.bfloat16),
            pltpu.VMEM((N_DEV, D, E_LOC), jnp.float32),
            pltpu.VMEM((N_DEV, T_LOC, 4), jnp.float32),
            pltpu.VMEM((2, D, H), jnp.float32),
            pltpu.VMEM((2, H, D), jnp.float32),
            pltpu.VMEM((N_DEV, T_LOC, D), jnp.float32),
            pltpu.VMEM((N_DEV - 1, T_LOC, D), jnp.bfloat16),
            pltpu.VMEM((N_DEV, T_LOC, D), jnp.bfloat16),
            pltpu.SemaphoreType.DMA((N_DEV - 1,)),
            pltpu.SemaphoreType.DMA((N_DEV,)),
            pltpu.SemaphoreType.DMA((N_DEV - 1,)),
            pltpu.SemaphoreType.DMA((N_DEV,)),
            pltpu.SemaphoreType.DMA((N_DEV - 1,)),
            pltpu.SemaphoreType.DMA((N_DEV,)),
            pltpu.SemaphoreType.DMA((N_DEV - 1,)),
            pltpu.SemaphoreType.DMA((N_DEV,)),
            pltpu.SemaphoreType.DMA((2,)),
            pltpu.SemaphoreType.DMA((2,)),
        ],
        compiler_params=pltpu.CompilerParams(
            collective_id=0, vmem_limit_bytes=64 * 1024 * 1024),
    )(x, router, W1, W2)
